# baseline (speedup 1.0000x reference)
import jax
import jax.numpy as jnp
from jax.experimental import pallas as pl

_NNODES = 208
_NCLASS = 64

_DN = (((1,), (1,)), ((), ()))


def _gcn(x_ref, adj_ref, w1_ref, b1_ref, w2_ref, b2_ref, h2_ref):
    f32 = jnp.float32
    adj = adj_ref[...]
    s1 = jnp.dot(x_ref[...], w1_ref[...], preferred_element_type=f32)
    h1 = jnp.maximum(jnp.dot(adj, s1, preferred_element_type=f32) + b1_ref[...], 0.0)
    s2 = jnp.dot(h1, w2_ref[...], preferred_element_type=f32)
    h2_ref[...] = jnp.maximum(
        jnp.dot(adj, s2, preferred_element_type=f32) + b2_ref[...], 0.0)


def _head(hflat_ref, fc1w_ref, fc1b_ref, fc2w_ref, fc2b_ref, fc3w_ref, fc3b_ref,
          out_ref):
    f32 = jnp.float32
    f1 = jax.lax.dot_general(hflat_ref[...], fc1w_ref[...], _DN,
                             preferred_element_type=f32)
    f1 = jnp.maximum(f1 + fc1b_ref[...], 0.0)
    f2 = jnp.sum(fc2w_ref[...] * f1, axis=1, keepdims=True)
    f2 = jnp.maximum(f2 + fc2b_ref[...], 0.0)
    f3 = jnp.sum(f2 * fc3w_ref[...], keepdims=True) + fc3b_ref[...]
    out_ref[...] = jax.nn.sigmoid(f3)


def kernel(x, adj, W1, b1, W2, b2, fc1_w, fc1_b, fc2_w, fc2_b, fc3_w, fc3_b):
    h2 = pl.pallas_call(
        _gcn,
        out_shape=jax.ShapeDtypeStruct((_NNODES, _NCLASS), jnp.float32),
    )(x, adj, W1, b1.reshape(1, -1), W2, b2.reshape(1, -1))
    hflat = h2.reshape(1, _NNODES * _NCLASS)
    out = pl.pallas_call(
        _head,
        out_shape=jax.ShapeDtypeStruct((1, 1), jnp.float32),
    )(hflat, fc1_w, fc1_b.reshape(1, -1), fc2_w, fc2_b.reshape(-1, 1),
      fc3_w.reshape(-1, 1), fc3_b.reshape(1, 1))
    return out.reshape(1)
